# baseline (speedup 1.0000x reference)
import math

import jax
import jax.numpy as jnp
import numpy as np
from jax import lax
from jax.experimental import pallas as pl
from jax.experimental.pallas import tpu as pltpu
from jax.experimental.pallas import tpu_sc as plsc

_NC, _NS, _L = 2, 16, 16
_NW = _NC * _NS

_B, _S, _D = 4, 2048, 1024
_NTOK = _B * _S
_CHUNK = 32
_NBLK = 2
_NCHUNK = _NBLK * _B
_NBUF = 2
_SCALE = math.sqrt(_D)


def _pos_encoding(seq_len, d_model):
    position = np.arange(seq_len, dtype=np.float32)[:, None]
    div_term = np.exp(
        np.arange(0, d_model, 2, dtype=np.float32)
        * (-math.log(10000.0) / d_model))
    pe = np.zeros((seq_len, d_model), dtype=np.float32)
    pe[:, 0::2] = np.sin(position * div_term)
    pe[:, 1::2] = np.cos(position * div_term)
    return pe


def _pe_bf16_words():
    import ml_dtypes
    pe = _pos_encoding(_S, _D)
    pe = pe.reshape(_S, _D // 32, 2, 16).transpose(0, 1, 3, 2).reshape(_S, _D)
    bf = pe.astype(ml_dtypes.bfloat16)
    return bf.reshape(_S, _D // 2, 2).view(np.int32).reshape(_S, _D // 2)


_PE_W = _pe_bf16_words()


@jax.jit
def _embed(x_flat, table, pe_w):
    mesh = plsc.VectorSubcoreMesh(
        core_axis_name="c", subcore_axis_name="s",
        num_cores=_NC, num_subcores=_NS)

    @pl.kernel(
        out_type=jax.ShapeDtypeStruct((_NTOK, _D), jnp.float32),
        mesh=mesh,
        scratch_types=[
            pltpu.VMEM((_NCHUNK * _CHUNK,), jnp.int32),
            pltpu.VMEM((_CHUNK, _D), jnp.float32),
            pltpu.VMEM((_CHUNK, _D), jnp.float32),
            pltpu.VMEM((_CHUNK, _D), jnp.float32),
            pltpu.VMEM((_CHUNK, _D // 2), jnp.int32),
            [pltpu.SemaphoreType.DMA] * _NBUF,
            [pltpu.SemaphoreType.DMA] * _NBUF,
            pltpu.SemaphoreType.DMA,
            pltpu.SemaphoreType.DMA,
        ],
    )
    def body(idx_hbm, table_hbm, pe_hbm, out_hbm,
             idx_v, rows0, rows1, pe_v, pe_wv, gsems, ssems, pesem, isem):
        cid = lax.axis_index("c")
        sid = lax.axis_index("s")
        wid = sid * _NC + cid
        rows = (rows0, rows1)

        def tok_base(j):
            blk, batch = divmod(j, _B)
            return batch * _S + blk * (_S // 2) + _CHUNK * wid

        idx_d = [
            pltpu.async_copy(
                idx_hbm.at[pl.ds(tok_base(j), _CHUNK)],
                idx_v.at[pl.ds(j * _CHUNK, _CHUNK)], isem)
            for j in range(_NCHUNK)
        ]
        for d in idx_d:
            d.wait()

        def load_pe_words(blk):
            base = blk * (_S // 2) + _CHUNK * wid
            return pltpu.async_copy(
                pe_hbm.at[pl.ds(base, _CHUNK)], pe_wv, pesem)

        def decode_pe():
            def row_dec(r, carry):
                for g in range(_D // 32):
                    w = pe_wv[r, pl.ds(g * 16, 16)]
                    pe_v[r, pl.ds(g * 32, 16)] = lax.bitcast_convert_type(
                        lax.shift_left(w, jnp.int32(16)), jnp.float32)
                    pe_v[r, pl.ds(g * 32 + 16, 16)] = lax.bitcast_convert_type(
                        lax.bitwise_and(w, jnp.int32(-65536)), jnp.float32)
                return carry
            lax.fori_loop(0, _CHUNK, row_dec, 0)

        def gather(j):
            return pltpu.async_copy(
                table_hbm.at[idx_v.at[pl.ds(j * _CHUNK, _CHUNK)]],
                rows[j % _NBUF], gsems[j % _NBUF])

        def fma(j):
            buf = rows[j % _NBUF]

            def row_fma(i, carry):
                r = i * 2
                for rr in (r, r + 1):
                    for g in range(_D // _L):
                        sl = pl.ds(g * _L, _L)
                        buf[rr, sl] = buf[rr, sl] * _SCALE + pe_v[rr, sl]
                return carry
            lax.fori_loop(0, _CHUNK // 2, row_fma, 0)

        def store(j):
            return pltpu.async_copy(
                rows[j % _NBUF], out_hbm.at[pl.ds(tok_base(j), _CHUNK)],
                ssems[j % _NBUF])

        pe_d = load_pe_words(0)
        g_d = [None] * _NCHUNK
        s_d = [None] * _NCHUNK
        g_d[0] = gather(0)
        pe_d.wait()
        decode_pe()
        pe_d = load_pe_words(1)
        for k in range(_NCHUNK):
            if k + 1 < _NCHUNK:
                if k - 1 >= 0:
                    s_d[k - 1].wait()
                g_d[k + 1] = gather(k + 1)
            g_d[k].wait()
            fma(k)
            s_d[k] = store(k)
            if k == _B - 1:
                pe_d.wait()
                decode_pe()
        for j in range(_NCHUNK - 2, _NCHUNK):
            s_d[j].wait()

    return body(x_flat, table, pe_w)


def kernel(x, table):
    pe_w = jnp.asarray(_PE_W)
    out = _embed(x.astype(jnp.int32).reshape(_NTOK), table, pe_w)
    return out.reshape(_B, _S, _D)


# speedup vs baseline: 1.1845x; 1.1845x over previous
import math

import jax
import jax.numpy as jnp
import numpy as np
from jax import lax
from jax.experimental import pallas as pl
from jax.experimental.pallas import tpu as pltpu
from jax.experimental.pallas import tpu_sc as plsc

_NC, _NS, _L = 2, 16, 16
_NW = _NC * _NS

_B, _S, _D = 4, 2048, 1024
_NTOK = _B * _S
_CHUNK = 32
_NBLK = 2
_NCHUNK = _NBLK * _B
_NBUF = 2
_SCALE = math.sqrt(_D)


def _pos_encoding(seq_len, d_model):
    position = np.arange(seq_len, dtype=np.float32)[:, None]
    div_term = np.exp(
        np.arange(0, d_model, 2, dtype=np.float32)
        * (-math.log(10000.0) / d_model))
    pe = np.zeros((seq_len, d_model), dtype=np.float32)
    pe[:, 0::2] = np.sin(position * div_term)
    pe[:, 1::2] = np.cos(position * div_term)
    return pe


def _pe_bf16_words():
    import ml_dtypes
    pe = _pos_encoding(_S, _D)
    pe = pe.reshape(_S, _D // 32, 2, 16).transpose(0, 1, 3, 2).reshape(_S, _D)
    bf = pe.astype(ml_dtypes.bfloat16)
    return bf.reshape(_S, _D // 2, 2).view(np.int32).reshape(_S, _D // 2)


_PE_W = _pe_bf16_words()


@jax.jit
def _embed(x_flat, table, pe_w):
    mesh = plsc.VectorSubcoreMesh(
        core_axis_name="c", subcore_axis_name="s",
        num_cores=_NC, num_subcores=_NS)

    @pl.kernel(
        out_type=jax.ShapeDtypeStruct((_NTOK, _D), jnp.float32),
        mesh=mesh,
        scratch_types=[
            pltpu.VMEM((_NCHUNK * _CHUNK,), jnp.int32),
            pltpu.VMEM((_CHUNK, _D), jnp.float32),
            pltpu.VMEM((_CHUNK, _D), jnp.float32),
            pltpu.VMEM((_CHUNK, _D), jnp.float32),
            pltpu.VMEM((_CHUNK, _D // 2), jnp.int32),
            [pltpu.SemaphoreType.DMA] * _NBUF,
            [pltpu.SemaphoreType.DMA] * _NBUF,
            pltpu.SemaphoreType.DMA,
            pltpu.SemaphoreType.DMA,
        ],
    )
    def body(idx_hbm, table_hbm, pe_hbm, out_hbm,
             idx_v, rows0, rows1, pe_v, pe_wv, gsems, ssems, pesem, isem):
        cid = lax.axis_index("c")
        sid = lax.axis_index("s")
        wid = sid * _NC + cid
        rows = (rows0, rows1)

        def tok_base(j):
            blk, batch = divmod(j, _B)
            return batch * _S + blk * (_S // 2) + _CHUNK * wid

        idx_d = [
            pltpu.async_copy(
                idx_hbm.at[pl.ds(tok_base(j), _CHUNK)],
                idx_v.at[pl.ds(j * _CHUNK, _CHUNK)], isem)
            for j in range(_NCHUNK)
        ]
        for d in idx_d:
            d.wait()

        def load_pe_words(blk):
            base = blk * (_S // 2) + _CHUNK * wid
            return pltpu.async_copy(
                pe_hbm.at[pl.ds(base, _CHUNK)], pe_wv, pesem)

        def decode_pe():
            def row_dec(r, carry):
                for g in range(_D // 32):
                    w = pe_wv[r, pl.ds(g * 16, 16)]
                    pe_v[r, pl.ds(g * 32, 16)] = lax.bitcast_convert_type(
                        lax.shift_left(w, jnp.int32(16)), jnp.float32)
                    pe_v[r, pl.ds(g * 32 + 16, 16)] = lax.bitcast_convert_type(
                        lax.bitwise_and(w, jnp.int32(-65536)), jnp.float32)
                return carry
            lax.fori_loop(0, _CHUNK, row_dec, 0)

        def gather(j):
            return pltpu.async_copy(
                table_hbm.at[idx_v.at[pl.ds(j * _CHUNK, _CHUNK)]],
                rows[j % _NBUF], gsems[j % _NBUF])

        def fma(j):
            buf = rows[j % _NBUF]

            @plsc.parallel_loop(0, _CHUNK, 1)
            def row_fma(r):
                for g in range(_D // _L):
                    sl = pl.ds(g * _L, _L)
                    buf[r, sl] = buf[r, sl] * _SCALE + pe_v[r, sl]

        def store(j):
            return pltpu.async_copy(
                rows[j % _NBUF], out_hbm.at[pl.ds(tok_base(j), _CHUNK)],
                ssems[j % _NBUF])

        pe_d = load_pe_words(0)
        g_d = [None] * _NCHUNK
        s_d = [None] * _NCHUNK
        g_d[0] = gather(0)
        pe_d.wait()
        decode_pe()
        pe_d = load_pe_words(1)
        for k in range(_NCHUNK):
            if k + 1 < _NCHUNK:
                if k - 1 >= 0:
                    s_d[k - 1].wait()
                g_d[k + 1] = gather(k + 1)
            g_d[k].wait()
            fma(k)
            s_d[k] = store(k)
            if k == _B - 1:
                pe_d.wait()
                decode_pe()
        for j in range(_NCHUNK - 2, _NCHUNK):
            s_d[j].wait()

    return body(x_flat, table, pe_w)


def kernel(x, table):
    pe_w = jnp.asarray(_PE_W)
    out = _embed(x.astype(jnp.int32).reshape(_NTOK), table, pe_w)
    return out.reshape(_B, _S, _D)


# speedup vs baseline: 1.2961x; 1.0942x over previous
import math

import jax
import jax.numpy as jnp
import numpy as np
from jax import lax
from jax.experimental import pallas as pl
from jax.experimental.pallas import tpu as pltpu
from jax.experimental.pallas import tpu_sc as plsc

_NC, _NS, _L = 2, 16, 16
_NW = _NC * _NS

_B, _S, _D = 4, 2048, 1024
_NTOK = _B * _S
_CHUNK = 32
_NBLK = 2
_NCHUNK = _NBLK * _B
_NBUF = 2
_SCALE = math.sqrt(_D)


def _pos_encoding(seq_len, d_model):
    position = np.arange(seq_len, dtype=np.float32)[:, None]
    div_term = np.exp(
        np.arange(0, d_model, 2, dtype=np.float32)
        * (-math.log(10000.0) / d_model))
    pe = np.zeros((seq_len, d_model), dtype=np.float32)
    pe[:, 0::2] = np.sin(position * div_term)
    pe[:, 1::2] = np.cos(position * div_term)
    return pe


def _pe_bf16_words():
    import ml_dtypes
    pe = _pos_encoding(_S, _D)
    pe = pe.reshape(_S, _D // 32, 2, 16).transpose(0, 1, 3, 2).reshape(_S, _D)
    bf = pe.astype(ml_dtypes.bfloat16)
    return bf.reshape(_S, _D // 2, 2).view(np.int32).reshape(_S, _D // 2)


_PE_W = _pe_bf16_words()


@jax.jit
def _embed(x_flat, table, pe_w):
    mesh = plsc.VectorSubcoreMesh(
        core_axis_name="c", subcore_axis_name="s",
        num_cores=_NC, num_subcores=_NS)

    @pl.kernel(
        out_type=jax.ShapeDtypeStruct((_NTOK, _D), jnp.float32),
        mesh=mesh,
        scratch_types=[
            pltpu.VMEM((_NCHUNK * _CHUNK,), jnp.int32),
            pltpu.VMEM((_CHUNK, _D), jnp.float32),
            pltpu.VMEM((_CHUNK, _D), jnp.float32),
            pltpu.VMEM((_CHUNK, _D), jnp.float32),
            pltpu.VMEM((_CHUNK, _D // 2), jnp.int32),
            [pltpu.SemaphoreType.DMA] * _NBUF,
            [pltpu.SemaphoreType.DMA] * _NBUF,
            pltpu.SemaphoreType.DMA,
            pltpu.SemaphoreType.DMA,
        ],
    )
    def body(idx_hbm, table_hbm, pe_hbm, out_hbm,
             idx_v, rows0, rows1, pe_v, pe_wv, gsems, ssems, pesem, isem):
        cid = lax.axis_index("c")
        sid = lax.axis_index("s")
        wid = sid * _NC + cid
        rows = (rows0, rows1)

        def tok_base(j):
            blk, batch = divmod(j, _B)
            return batch * _S + blk * (_S // 2) + _CHUNK * wid

        idx_d = [
            pltpu.async_copy(
                idx_hbm.at[pl.ds(tok_base(j), _CHUNK)],
                idx_v.at[pl.ds(j * _CHUNK, _CHUNK)], isem)
            for j in range(_NCHUNK)
        ]
        for d in idx_d:
            d.wait()

        def load_pe_words(blk):
            base = blk * (_S // 2) + _CHUNK * wid
            return pltpu.async_copy(
                pe_hbm.at[pl.ds(base, _CHUNK)], pe_wv, pesem)

        def decode_pe():
            @plsc.parallel_loop(0, _CHUNK, 1)
            def row_dec(r):
                for g in range(_D // 32):
                    w = pe_wv[r, pl.ds(g * 16, 16)]
                    pe_v[r, pl.ds(g * 32, 16)] = lax.bitcast_convert_type(
                        lax.shift_left(w, jnp.int32(16)), jnp.float32)
                    pe_v[r, pl.ds(g * 32 + 16, 16)] = lax.bitcast_convert_type(
                        lax.bitwise_and(w, jnp.int32(-65536)), jnp.float32)

        def gather(j):
            return pltpu.async_copy(
                table_hbm.at[idx_v.at[pl.ds(j * _CHUNK, _CHUNK)]],
                rows[j % _NBUF], gsems[j % _NBUF])

        def fma(j):
            buf = rows[j % _NBUF]

            @plsc.parallel_loop(0, _CHUNK, 1)
            def row_fma(r):
                for g in range(_D // _L):
                    sl = pl.ds(g * _L, _L)
                    buf[r, sl] = buf[r, sl] * _SCALE + pe_v[r, sl]

        def store(j):
            return pltpu.async_copy(
                rows[j % _NBUF], out_hbm.at[pl.ds(tok_base(j), _CHUNK)],
                ssems[j % _NBUF])

        pe_d = load_pe_words(0)
        g_d = [None] * _NCHUNK
        s_d = [None] * _NCHUNK
        g_d[0] = gather(0)
        pe_d.wait()
        decode_pe()
        pe_d = load_pe_words(1)
        for k in range(_NCHUNK):
            if k + 1 < _NCHUNK:
                if k - 1 >= 0:
                    s_d[k - 1].wait()
                g_d[k + 1] = gather(k + 1)
            g_d[k].wait()
            fma(k)
            s_d[k] = store(k)
            if k == _B - 1:
                pe_d.wait()
                decode_pe()
        for j in range(_NCHUNK - 2, _NCHUNK):
            s_d[j].wait()

    return body(x_flat, table, pe_w)


def kernel(x, table):
    pe_w = jnp.asarray(_PE_W)
    out = _embed(x.astype(jnp.int32).reshape(_NTOK), table, pe_w)
    return out.reshape(_B, _S, _D)
